# baseline (speedup 1.0000x reference)
import functools

import jax
import jax.numpy as jnp
from jax import lax
from jax.experimental import pallas as pl
from jax.experimental.pallas import tpu as pltpu
from jax.experimental.pallas import tpu_sc as plsc

MAX_SEQ_LEN = 8192
D_MODEL = 1024
B = 8192

_info = plsc.get_sparse_core_info()
_NC, _NS = _info.num_cores, _info.num_subcores
_NW = _NC * _NS
_BPW = B // _NW
_CHUNK = 16
_NBUF = 4
_NCHUNK = _BPW // _CHUNK


def _gather_body(pe_hbm, pos_hbm, out_hbm, idx_v, *bufs_and_sems):
    rows = bufs_and_sems[:_NBUF]
    gsem = bufs_and_sems[_NBUF:2 * _NBUF]
    wsem = bufs_and_sems[2 * _NBUF:3 * _NBUF]
    wid = lax.axis_index("s") * _NC + lax.axis_index("c")
    base = wid * _BPW

    pltpu.sync_copy(pos_hbm.at[pl.ds(base, _BPW)], idx_v)

    def start_gather(i, b):
        pltpu.async_copy(
            pe_hbm.at[idx_v.at[pl.ds(i * _CHUNK, _CHUNK)]], rows[b], gsem[b]
        )

    for b in range(_NBUF):
        start_gather(b, b)
    writes = {}
    for i in range(_NCHUNK):
        b = i % _NBUF
        pltpu.make_async_copy(
            pe_hbm.at[idx_v.at[pl.ds(i * _CHUNK, _CHUNK)]], rows[b], gsem[b]
        ).wait()
        writes[i] = pltpu.async_copy(
            rows[b], out_hbm.at[pl.ds(base + i * _CHUNK, _CHUNK)], wsem[b]
        )
        nxt = i + _NBUF
        if nxt < _NCHUNK:
            writes[i].wait()
            start_gather(nxt, b)
    for i in range(max(0, _NCHUNK - _NBUF), _NCHUNK):
        writes[i].wait()


@jax.jit
def _gather(pe, positions):
    mesh = plsc.VectorSubcoreMesh(core_axis_name="c", subcore_axis_name="s")
    return pl.kernel(
        _gather_body,
        mesh=mesh,
        out_type=jax.ShapeDtypeStruct((B, D_MODEL), jnp.float32),
        scratch_types=(
            [pltpu.VMEM((_BPW,), jnp.int32)]
            + [pltpu.VMEM((_CHUNK, D_MODEL), jnp.float32) for _ in range(_NBUF)]
            + [pltpu.SemaphoreType.DMA for _ in range(2 * _NBUF)]
        ),
    )(pe, positions)


def kernel(pe, positions):
    return _gather(pe, positions.astype(jnp.int32))


# speedup vs baseline: 1.0208x; 1.0208x over previous
import functools

import jax
import jax.numpy as jnp
from jax import lax
from jax.experimental import pallas as pl
from jax.experimental.pallas import tpu as pltpu
from jax.experimental.pallas import tpu_sc as plsc

MAX_SEQ_LEN = 8192
D_MODEL = 1024
B = 8192

_info = plsc.get_sparse_core_info()
_NC, _NS = _info.num_cores, _info.num_subcores
_NW = _NC * _NS
_BPW = B // _NW
_CHUNK = 16
_NBUF = 7
_NCHUNK = _BPW // _CHUNK


def _gather_body(pe_hbm, pos_hbm, out_hbm, idx_v, *bufs_and_sems):
    rows = bufs_and_sems[:_NBUF]
    gsem = bufs_and_sems[_NBUF:2 * _NBUF]
    wsem = bufs_and_sems[2 * _NBUF:3 * _NBUF]
    wid = lax.axis_index("s") * _NC + lax.axis_index("c")
    base = wid * _BPW

    pltpu.sync_copy(pos_hbm.at[pl.ds(base, _BPW)], idx_v)

    def start_gather(i, b):
        pltpu.async_copy(
            pe_hbm.at[idx_v.at[pl.ds(i * _CHUNK, _CHUNK)]], rows[b], gsem[b]
        )

    for b in range(_NBUF):
        start_gather(b, b)
    writes = {}
    for i in range(_NCHUNK):
        b = i % _NBUF
        pltpu.make_async_copy(
            pe_hbm.at[idx_v.at[pl.ds(i * _CHUNK, _CHUNK)]], rows[b], gsem[b]
        ).wait()
        writes[i] = pltpu.async_copy(
            rows[b], out_hbm.at[pl.ds(base + i * _CHUNK, _CHUNK)], wsem[b]
        )
        nxt = i + _NBUF
        if nxt < _NCHUNK:
            writes[i].wait()
            start_gather(nxt, b)
    for i in range(max(0, _NCHUNK - _NBUF), _NCHUNK):
        writes[i].wait()


@jax.jit
def _gather(pe, positions):
    mesh = plsc.VectorSubcoreMesh(core_axis_name="c", subcore_axis_name="s")
    return pl.kernel(
        _gather_body,
        mesh=mesh,
        out_type=jax.ShapeDtypeStruct((B, D_MODEL), jnp.float32),
        scratch_types=(
            [pltpu.VMEM((_BPW,), jnp.int32)]
            + [pltpu.VMEM((_CHUNK, D_MODEL), jnp.float32) for _ in range(_NBUF)]
            + [pltpu.SemaphoreType.DMA for _ in range(2 * _NBUF)]
        ),
    )(pe, positions)


def kernel(pe, positions):
    return _gather(pe, positions.astype(jnp.int32))
